# baseline (speedup 1.0000x reference)
import functools

import jax
import jax.numpy as jnp
from jax import lax
from jax.experimental import pallas as pl
from jax.experimental.pallas import tpu as pltpu
from jax.experimental.pallas import tpu_sc as plsc

_NC = 2
_NS = 16
_NW = _NC * _NS
_GROUP = 8
_LANES = 16


def _sc_gather_sum(idx_pad, table, n_valid):
    vocab, d = table.shape
    mesh = plsc.VectorSubcoreMesh(core_axis_name="c", subcore_axis_name="s")

    def body(idx_hbm, table_hbm, out_hbm, idx_v, rows_v, acc_v, sem):
        wid = lax.axis_index("s") * _NC + lax.axis_index("c")
        base = wid * _GROUP
        pltpu.sync_copy(idx_hbm.at[pl.ds(base, _GROUP)], idx_v)
        pltpu.async_copy(table_hbm.at[idx_v], rows_v, sem).wait()
        for c in range(d // _LANES):
            sl = pl.ds(c * _LANES, _LANES)
            acc = jnp.zeros((_LANES,), jnp.float32)
            for r in range(_GROUP):
                w_r = jnp.where(base + r < n_valid, 1.0, 0.0)
                acc = acc + rows_v[r, sl] * w_r
            acc_v[sl] = acc
        pltpu.sync_copy(acc_v, out_hbm.at[wid])

    run = pl.kernel(
        body,
        out_type=jax.ShapeDtypeStruct((_NW, d), jnp.float32),
        mesh=mesh,
        scratch_types=[
            pltpu.VMEM((_GROUP,), jnp.int32),
            pltpu.VMEM((_GROUP, d), jnp.float32),
            pltpu.VMEM((d,), jnp.float32),
            pltpu.SemaphoreType.DMA,
        ],
    )
    return run(idx_pad, table)


def _a_body(nta, ntb, tv, vocab, pT_ref, wpT_ref, bpT_ref, wo1_ref, wo2_ref,
            bo1_ref, bo2_ref, lga_ref, lgb_ref, lse_ref, h_sc, m_ref, s_ref):
    i = pl.program_id(0)

    @pl.when(i == 0)
    def _init():
        e = jnp.sum(pT_ref[...], axis=1, keepdims=True)
        e8 = jnp.broadcast_to(e, (e.shape[0], 8))
        h = jnp.dot(wpT_ref[...], e8, preferred_element_type=jnp.float32)
        h_sc[...] = jnp.maximum(h + bpT_ref[...], 0.0)
        m_ref[0] = -jnp.inf
        s_ref[0] = 0.0

    lt1 = jnp.dot(wo1_ref[...], h_sc[...], preferred_element_type=jnp.float32)
    lr1 = jnp.transpose(lt1)[0:1, :] + bo1_ref[...]
    lga_ref[...] = lr1
    m1 = jnp.maximum(m_ref[0], jnp.max(lr1))
    scale1 = jnp.max(jnp.exp(jnp.full((1, 128), m_ref[0] - m1)))
    s_ref[0] = s_ref[0] * scale1 + jnp.sum(jnp.exp(lr1 - m1))
    m_ref[0] = m1

    @pl.when(i < ntb)
    def _stream2():
        lt2 = jnp.dot(wo2_ref[...], h_sc[...],
                      preferred_element_type=jnp.float32)
        lr2 = jnp.transpose(lt2)[0:1, :] + bo2_ref[...]
        lgb_ref[...] = lr2
        col = (nta + i) * tv + lax.broadcasted_iota(jnp.int32, (1, tv), 1)
        lm2 = jnp.where(col < vocab, lr2, -jnp.inf)
        m2 = jnp.maximum(m_ref[0], jnp.max(lm2))
        scale2 = jnp.max(jnp.exp(jnp.full((1, 128), m_ref[0] - m2)))
        s_ref[0] = s_ref[0] * scale2 + jnp.sum(jnp.exp(lm2 - m2))
        m_ref[0] = m2

    @pl.when(i == nta - 1)
    def _finish():
        lse_ref[...] = jnp.full((1, 1), m_ref[0]) + jnp.log(
            jnp.full((1, 1), s_ref[0]))


def _tc_logits_lse(pT, wpT, bpT, woT, bo2):
    vocab, d = woT.shape
    tv = 8192
    nt = -(-vocab // tv)
    nta = -(-nt // 2)
    ntb = nt - nta

    return pl.pallas_call(
        functools.partial(_a_body, nta, ntb, tv, vocab),
        grid=(nta,),
        in_specs=[
            pl.BlockSpec((d, _NW), lambda i: (0, 0)),
            pl.BlockSpec((d, d), lambda i: (0, 0)),
            pl.BlockSpec((d, 1), lambda i: (0, 0)),
            pl.BlockSpec((tv, d), lambda i: (i, 0)),
            pl.BlockSpec((tv, d), lambda i: (jnp.minimum(nta + i, nt - 1), 0)),
            pl.BlockSpec((1, tv), lambda i: (0, i)),
            pl.BlockSpec((1, tv), lambda i: (0, jnp.minimum(nta + i, nt - 1))),
        ],
        out_specs=[
            pl.BlockSpec((1, tv), lambda i: (0, i)),
            pl.BlockSpec((1, tv), lambda i: (0, jnp.minimum(i, ntb - 1))),
            pl.BlockSpec((1, 1), lambda i: (0, 0)),
        ],
        out_shape=[
            jax.ShapeDtypeStruct((1, nta * tv), jnp.float32),
            jax.ShapeDtypeStruct((1, vocab - nta * tv), jnp.float32),
            jax.ShapeDtypeStruct((1, 1), jnp.float32),
        ],
        scratch_shapes=[
            pltpu.VMEM((d, 8), jnp.float32),
            pltpu.SMEM((1,), jnp.float32),
            pltpu.SMEM((1,), jnp.float32),
        ],
        compiler_params=pltpu.CompilerParams(
            dimension_semantics=("arbitrary",),
        ),
    )(pT, wpT, bpT, woT, woT, bo2, bo2)


def _b_body(wa, lga_ref, lgb_ref, lse_ref, out_ref):
    lse = lse_ref[...]
    out_ref[:, :wa] = lga_ref[...] - lse
    out_ref[:, wa:] = lgb_ref[...] - lse


def _tc_subtract(lga, lgb, lse):
    wa = lga.shape[1]
    vocab = wa + lgb.shape[1]
    return pl.pallas_call(
        functools.partial(_b_body, wa),
        in_specs=[
            pl.BlockSpec((1, wa), lambda: (0, 0)),
            pl.BlockSpec((1, vocab - wa), lambda: (0, 0)),
            pl.BlockSpec((1, 1), lambda: (0, 0)),
        ],
        out_specs=pl.BlockSpec((1, vocab), lambda: (0, 0)),
        out_shape=jax.ShapeDtypeStruct((1, vocab), jnp.float32),
    )(lga, lgb, lse)


def kernel(inputs, table, W_proj, b_proj, W_out, b_out):
    n = inputs.shape[0]
    idx = inputs.astype(jnp.int32)
    n_pad = _NW * _GROUP
    idx_pad = jnp.pad(idx, (0, n_pad - n))
    partials = _sc_gather_sum(idx_pad, table, n)
    lga, lgb, lse = _tc_logits_lse(
        partials.T,
        W_proj.T,
        b_proj.reshape(-1, 1),
        W_out.T,
        b_out.reshape(1, -1),
    )
    return _tc_subtract(lga, lgb, lse)


# speedup vs baseline: 1.0433x; 1.0433x over previous
import functools

import jax
import jax.numpy as jnp
from jax import lax
from jax.experimental import pallas as pl
from jax.experimental.pallas import tpu as pltpu
from jax.experimental.pallas import tpu_sc as plsc

_NC = 2
_NS = 16
_NW = _NC * _NS
_GROUP = 8
_LANES = 16


def _sc_gather_sum(idx_pad, table, n_valid):
    vocab, d = table.shape
    mesh = plsc.VectorSubcoreMesh(core_axis_name="c", subcore_axis_name="s")

    def body(idx_hbm, table_hbm, out_hbm, idx_v, rows_v, acc_v, sem):
        wid = lax.axis_index("s") * _NC + lax.axis_index("c")
        base = wid * _GROUP
        pltpu.sync_copy(idx_hbm.at[pl.ds(base, _GROUP)], idx_v)
        pltpu.async_copy(table_hbm.at[idx_v], rows_v, sem).wait()
        for c in range(d // _LANES):
            sl = pl.ds(c * _LANES, _LANES)
            acc = jnp.zeros((_LANES,), jnp.float32)
            for r in range(_GROUP):
                w_r = jnp.where(base + r < n_valid, 1.0, 0.0)
                acc = acc + rows_v[r, sl] * w_r
            acc_v[sl] = acc
        pltpu.sync_copy(acc_v, out_hbm.at[wid])

    run = pl.kernel(
        body,
        out_type=jax.ShapeDtypeStruct((_NW, d), jnp.float32),
        mesh=mesh,
        scratch_types=[
            pltpu.VMEM((_GROUP,), jnp.int32),
            pltpu.VMEM((_GROUP, d), jnp.float32),
            pltpu.VMEM((d,), jnp.float32),
            pltpu.SemaphoreType.DMA,
        ],
    )
    return run(idx_pad, table)


_TV = 4096
_NSTREAM = 4


def _stream_counts(vocab):
    nt = -(-vocab // _TV)
    base, rem = divmod(nt, _NSTREAM)
    cnts = [base + (1 if k < rem else 0) for k in range(_NSTREAM)]
    offs = [sum(cnts[:k]) for k in range(_NSTREAM)]
    return nt, cnts, offs


def _a_body(cnts, offs, nt, vocab, *refs):
    tv = _TV
    s_n = _NSTREAM
    pT_ref, wpT_ref, bpT_ref = refs[0:3]
    wo_refs = refs[3:3 + s_n]
    bo_refs = refs[3 + s_n:3 + 2 * s_n]
    lg_refs = refs[3 + 2 * s_n:3 + 3 * s_n]
    lse_ref = refs[3 + 3 * s_n]
    h_sc, m_ref, s_ref = refs[3 + 3 * s_n + 1:]
    i = pl.program_id(0)

    @pl.when(i == 0)
    def _init():
        e = jnp.sum(pT_ref[...], axis=1, keepdims=True)
        e8 = jnp.broadcast_to(e, (e.shape[0], 8))
        h = jnp.dot(wpT_ref[...], e8, preferred_element_type=jnp.float32)
        h_sc[...] = jnp.maximum(h + bpT_ref[...], 0.0)
        m_ref[0] = -jnp.inf
        s_ref[0] = 0.0

    for k in range(s_n):
        @pl.when(i < cnts[k])
        def _stream(k=k):
            lt = jnp.dot(wo_refs[k][...], h_sc[...],
                         preferred_element_type=jnp.float32)
            lr = jnp.transpose(lt)[0:1, :] + bo_refs[k][...]
            lg_refs[k][...] = lr
            col = (offs[k] + i) * tv + lax.broadcasted_iota(
                jnp.int32, (1, tv), 1)
            lm = jnp.where(col < vocab, lr, -jnp.inf)
            m_new = jnp.maximum(m_ref[0], jnp.max(lm))
            scale = jnp.max(jnp.exp(jnp.full((1, 128), m_ref[0] - m_new)))
            s_ref[0] = s_ref[0] * scale + jnp.sum(jnp.exp(lm - m_new))
            m_ref[0] = m_new

    @pl.when(i == cnts[0] - 1)
    def _finish():
        lse_ref[...] = jnp.full((1, 1), m_ref[0]) + jnp.log(
            jnp.full((1, 1), s_ref[0]))


def _tc_logits_lse(pT, wpT, bpT, woT, bo2):
    vocab, d = woT.shape
    tv = _TV
    nt, cnts, offs = _stream_counts(vocab)

    def _wo_spec(k):
        return pl.BlockSpec(
            (tv, d),
            lambda i: (jnp.minimum(offs[k] + i, offs[k] + cnts[k] - 1), 0))

    def _bo_spec(k):
        return pl.BlockSpec(
            (1, tv),
            lambda i: (0, jnp.minimum(offs[k] + i, offs[k] + cnts[k] - 1)))

    def _lg_spec(k):
        return pl.BlockSpec(
            (1, tv), lambda i: (0, jnp.minimum(i, cnts[k] - 1)))

    widths = [
        min((offs[k] + cnts[k]) * tv, vocab) - offs[k] * tv
        for k in range(_NSTREAM)
    ]
    return pl.pallas_call(
        functools.partial(_a_body, cnts, offs, nt, vocab),
        grid=(cnts[0],),
        in_specs=(
            [
                pl.BlockSpec((d, _NW), lambda i: (0, 0)),
                pl.BlockSpec((d, d), lambda i: (0, 0)),
                pl.BlockSpec((d, 1), lambda i: (0, 0)),
            ]
            + [_wo_spec(k) for k in range(_NSTREAM)]
            + [_bo_spec(k) for k in range(_NSTREAM)]
        ),
        out_specs=(
            [_lg_spec(k) for k in range(_NSTREAM)]
            + [pl.BlockSpec((1, 1), lambda i: (0, 0))]
        ),
        out_shape=(
            [jax.ShapeDtypeStruct((1, w), jnp.float32) for w in widths]
            + [jax.ShapeDtypeStruct((1, 1), jnp.float32)]
        ),
        scratch_shapes=[
            pltpu.VMEM((d, 8), jnp.float32),
            pltpu.SMEM((1,), jnp.float32),
            pltpu.SMEM((1,), jnp.float32),
        ],
        compiler_params=pltpu.CompilerParams(
            dimension_semantics=("arbitrary",),
        ),
    )(pT, wpT, bpT, *([woT] * _NSTREAM), *([bo2] * _NSTREAM))


def _b_body(widths, *refs):
    lg_refs = refs[:len(widths)]
    lse_ref = refs[len(widths)]
    out_ref = refs[len(widths) + 1]
    lse = lse_ref[...]
    o = 0
    for k, w in enumerate(widths):
        out_ref[:, o:o + w] = lg_refs[k][...] - lse
        o += w


def _tc_subtract(lgs, lse):
    widths = [x.shape[1] for x in lgs]
    vocab = sum(widths)
    return pl.pallas_call(
        functools.partial(_b_body, widths),
        in_specs=(
            [pl.BlockSpec((1, w), lambda: (0, 0)) for w in widths]
            + [pl.BlockSpec((1, 1), lambda: (0, 0))]
        ),
        out_specs=pl.BlockSpec((1, vocab), lambda: (0, 0)),
        out_shape=jax.ShapeDtypeStruct((1, vocab), jnp.float32),
    )(*lgs, lse)


def kernel(inputs, table, W_proj, b_proj, W_out, b_out):
    n = inputs.shape[0]
    idx = inputs.astype(jnp.int32)
    n_pad = _NW * _GROUP
    idx_pad = jnp.pad(idx, (0, n_pad - n))
    partials = _sc_gather_sum(idx_pad, table, n)
    *lgs, lse = _tc_logits_lse(
        partials.T,
        W_proj.T,
        b_proj.reshape(-1, 1),
        W_out.T,
        b_out.reshape(1, -1),
    )
    return _tc_subtract(lgs, lse)


# speedup vs baseline: 1.0992x; 1.0536x over previous
import functools

import jax
import jax.numpy as jnp
from jax import lax
from jax.experimental import pallas as pl
from jax.experimental.pallas import tpu as pltpu
from jax.experimental.pallas import tpu_sc as plsc

_NC = 2
_NS = 16
_NW = _NC * _NS
_GROUP = 8
_LANES = 16

_TV = 8192
_NBUF = 4


def _sc_gather_sum(idx_pad, table, n_valid):
    vocab, d = table.shape
    mesh = plsc.VectorSubcoreMesh(core_axis_name="c", subcore_axis_name="s")

    def body(idx_hbm, table_hbm, out_hbm, idx_v, rows_v, acc_v, sem):
        wid = lax.axis_index("s") * _NC + lax.axis_index("c")
        base = wid * _GROUP
        pltpu.sync_copy(idx_hbm.at[pl.ds(base, _GROUP)], idx_v)
        pltpu.async_copy(table_hbm.at[idx_v], rows_v, sem).wait()
        for c in range(d // _LANES):
            sl = pl.ds(c * _LANES, _LANES)
            acc = jnp.zeros((_LANES,), jnp.float32)
            for r in range(_GROUP):
                w_r = jnp.where(base + r < n_valid, 1.0, 0.0)
                acc = acc + rows_v[r, sl] * w_r
            acc_v[sl] = acc
        pltpu.sync_copy(acc_v, out_hbm.at[wid])

    run = pl.kernel(
        body,
        out_type=jax.ShapeDtypeStruct((_NW, d), jnp.float32),
        mesh=mesh,
        scratch_types=[
            pltpu.VMEM((_GROUP,), jnp.int32),
            pltpu.VMEM((_GROUP, d), jnp.float32),
            pltpu.VMEM((d,), jnp.float32),
            pltpu.SemaphoreType.DMA,
        ],
    )
    return run(idx_pad, table)


def _scalar_exp(x):
    return jnp.max(jnp.exp(jnp.full((1, 128), x)))


def _scalar_log(x):
    return jnp.max(jnp.log(jnp.full((1, 128), x)))


def _a_body(nt, vocab, pT_ref, wpT_ref, bpT_ref, wo_hbm, bo_ref, out_ref,
            *scr):
    bufs = scr[:_NBUF]
    sems = scr[_NBUF:]
    tv = _TV

    def tile_copy(t):
        sz = min(vocab - t * tv, tv)
        return pltpu.make_async_copy(
            wo_hbm.at[pl.ds(t * tv, sz)],
            bufs[t % _NBUF].at[pl.ds(0, sz)],
            sems[t % _NBUF],
        )

    for t in range(min(_NBUF - 1, nt)):
        tile_copy(t).start()

    e = jnp.sum(pT_ref[...], axis=1, keepdims=True)
    e8 = jnp.broadcast_to(e, (e.shape[0], 8))
    h = jnp.dot(wpT_ref[...], e8, preferred_element_type=jnp.float32)
    h = jnp.maximum(h + bpT_ref[...], 0.0)

    m = jnp.float32(-jnp.inf)
    s = jnp.float32(0.0)
    for t in range(nt):
        if t + _NBUF - 1 < nt:
            tile_copy(t + _NBUF - 1).start()
        tile_copy(t).wait()
        sz = min(vocab - t * tv, tv)
        wo = bufs[t % _NBUF][pl.ds(0, sz), :]
        lt = jnp.dot(wo, h, preferred_element_type=jnp.float32)
        lr = jnp.transpose(lt)[0:1, :] + bo_ref[:, t * tv:t * tv + sz]
        out_ref[:, t * tv:t * tv + sz] = lr
        m_new = jnp.maximum(m, jnp.max(lr))
        s = s * _scalar_exp(m - m_new) + jnp.sum(jnp.exp(lr - m_new))
        m = m_new

    lse = m + _scalar_log(s)
    out_ref[...] = out_ref[...] - lse


def _tc_mlp_logsoftmax(pT, wpT, bpT, woT, bo2):
    vocab, d = woT.shape
    nt = -(-vocab // _TV)

    return pl.pallas_call(
        functools.partial(_a_body, nt, vocab),
        in_specs=[
            pl.BlockSpec((d, _NW), lambda: (0, 0)),
            pl.BlockSpec((d, d), lambda: (0, 0)),
            pl.BlockSpec((d, 1), lambda: (0, 0)),
            pl.BlockSpec(memory_space=pl.ANY),
            pl.BlockSpec((1, vocab), lambda: (0, 0)),
        ],
        out_specs=pl.BlockSpec((1, vocab), lambda: (0, 0)),
        out_shape=jax.ShapeDtypeStruct((1, vocab), jnp.float32),
        scratch_shapes=(
            [pltpu.VMEM((_TV, d), jnp.float32) for _ in range(_NBUF)]
            + [pltpu.SemaphoreType.DMA for _ in range(_NBUF)]
        ),
    )(pT, wpT, bpT, woT, bo2)


def kernel(inputs, table, W_proj, b_proj, W_out, b_out):
    n = inputs.shape[0]
    idx = inputs.astype(jnp.int32)
    n_pad = _NW * _GROUP
    idx_pad = jnp.pad(idx, (0, n_pad - n))
    partials = _sc_gather_sum(idx_pad, table, n)
    return _tc_mlp_logsoftmax(
        partials.T,
        W_proj.T,
        b_proj.reshape(-1, 1),
        W_out.T,
        b_out.reshape(1, -1),
    )


# speedup vs baseline: 1.1571x; 1.0526x over previous
import functools

import jax
import jax.numpy as jnp
from jax import lax
from jax.experimental import pallas as pl
from jax.experimental.pallas import tpu as pltpu
from jax.experimental.pallas import tpu_sc as plsc

_NC = 1
_NS = 16
_NW = _NC * _NS
_GROUP = 16
_LANES = 16

_TV = 16384
_NBUF = 3


def _sc_gather_sum(idx_pad, table, n_valid):
    vocab, d = table.shape
    mesh = plsc.VectorSubcoreMesh(
        core_axis_name="c", subcore_axis_name="s", num_cores=_NC)

    def body(idx_hbm, table_hbm, out_hbm, idx_v, rows_v, acc_v, sem):
        wid = lax.axis_index("s") * _NC + lax.axis_index("c")
        base = wid * _GROUP
        pltpu.sync_copy(idx_hbm.at[pl.ds(base, _GROUP)], idx_v)
        pltpu.async_copy(table_hbm.at[idx_v], rows_v, sem).wait()
        for c in range(d // _LANES):
            sl = pl.ds(c * _LANES, _LANES)
            acc = jnp.zeros((_LANES,), jnp.float32)
            for r in range(_GROUP):
                w_r = jnp.where(base + r < n_valid, 1.0, 0.0)
                acc = acc + rows_v[r, sl] * w_r
            acc_v[sl] = acc
        pltpu.sync_copy(acc_v, out_hbm.at[wid])

    run = pl.kernel(
        body,
        out_type=jax.ShapeDtypeStruct((_NW, d), jnp.float32),
        mesh=mesh,
        scratch_types=[
            pltpu.VMEM((_GROUP,), jnp.int32),
            pltpu.VMEM((_GROUP, d), jnp.float32),
            pltpu.VMEM((d,), jnp.float32),
            pltpu.SemaphoreType.DMA,
        ],
    )
    return run(idx_pad, table)


def _scalar_exp(x):
    return jnp.max(jnp.exp(jnp.full((1, 128), x)))


def _scalar_log(x):
    return jnp.max(jnp.log(jnp.full((1, 128), x)))


def _a_body(nt, vocab, pT_ref, wpT_ref, bpT_ref, wo_hbm, bo_ref, out_ref,
            *scr):
    bufs = scr[:_NBUF]
    sems = scr[_NBUF:]
    tv = _TV

    def tile_copy(t):
        sz = min(vocab - t * tv, tv)
        return pltpu.make_async_copy(
            wo_hbm.at[pl.ds(t * tv, sz)],
            bufs[t % _NBUF].at[pl.ds(0, sz)],
            sems[t % _NBUF],
        )

    for t in range(min(_NBUF - 1, nt)):
        tile_copy(t).start()

    e = jnp.sum(pT_ref[...], axis=1, keepdims=True)
    e8 = jnp.broadcast_to(e, (e.shape[0], 8))
    h = jnp.dot(wpT_ref[...], e8, preferred_element_type=jnp.float32)
    h = jnp.maximum(h + bpT_ref[...], 0.0)

    m = jnp.float32(-jnp.inf)
    s = jnp.float32(0.0)
    for t in range(nt):
        if t + _NBUF - 1 < nt:
            tile_copy(t + _NBUF - 1).start()
        tile_copy(t).wait()
        sz = min(vocab - t * tv, tv)
        wo = bufs[t % _NBUF][pl.ds(0, sz), :]
        lt = jnp.dot(wo, h, preferred_element_type=jnp.float32)
        lr = jnp.transpose(lt)[0:1, :] + bo_ref[:, t * tv:t * tv + sz]
        out_ref[:, t * tv:t * tv + sz] = lr
        m_new = jnp.maximum(m, jnp.max(lr))
        s = s * _scalar_exp(m - m_new) + jnp.sum(jnp.exp(lr - m_new))
        m = m_new

    lse = m + _scalar_log(s)
    out_ref[...] = out_ref[...] - lse


def _tc_mlp_logsoftmax(pT, wpT, bpT, woT, bo2):
    vocab, d = woT.shape
    nt = -(-vocab // _TV)

    return pl.pallas_call(
        functools.partial(_a_body, nt, vocab),
        in_specs=[
            pl.BlockSpec((d, _NW), lambda: (0, 0)),
            pl.BlockSpec((d, d), lambda: (0, 0)),
            pl.BlockSpec((d, 1), lambda: (0, 0)),
            pl.BlockSpec(memory_space=pl.ANY),
            pl.BlockSpec((1, vocab), lambda: (0, 0)),
        ],
        out_specs=pl.BlockSpec((1, vocab), lambda: (0, 0)),
        out_shape=jax.ShapeDtypeStruct((1, vocab), jnp.float32),
        scratch_shapes=(
            [pltpu.VMEM((_TV, d), jnp.float32) for _ in range(_NBUF)]
            + [pltpu.SemaphoreType.DMA for _ in range(_NBUF)]
        ),
    )(pT, wpT, bpT, woT, bo2)


def kernel(inputs, table, W_proj, b_proj, W_out, b_out):
    n = inputs.shape[0]
    idx = inputs.astype(jnp.int32)
    n_pad = _NW * _GROUP
    idx_pad = jnp.pad(idx, (0, n_pad - n))
    partials = _sc_gather_sum(idx_pad, table, n)
    return _tc_mlp_logsoftmax(
        partials.T,
        W_proj.T,
        b_proj.reshape(-1, 1),
        W_out.T,
        b_out.reshape(1, -1),
    )


# speedup vs baseline: 1.2353x; 1.0676x over previous
import functools

import jax
import jax.numpy as jnp
from jax import lax
from jax.experimental import pallas as pl
from jax.experimental.pallas import tpu as pltpu
from jax.experimental.pallas import tpu_sc as plsc

_NC = 1
_NS = 16
_NW = _NC * _NS
_GROUP = 16
_LANES = 16

_TV = 16384
_NBUF = 4


def _sc_gather_sum(idx, table):
    n = idx.shape[0]
    vocab, d = table.shape
    mesh = plsc.VectorSubcoreMesh(
        core_axis_name="c", subcore_axis_name="s", num_cores=_NC)

    def body(idx_hbm, table_hbm, out_hbm, idx_v, rows_v, acc_v, sem):
        wid = lax.axis_index("s") * _NC + lax.axis_index("c")
        base = wid * _GROUP
        off1 = jnp.minimum(base, n - 8)
        off2 = jnp.minimum(base + 8, n - 8)
        pltpu.sync_copy(idx_hbm.at[pl.ds(off1, 8)], idx_v.at[pl.ds(0, 8)])
        pltpu.sync_copy(idx_hbm.at[pl.ds(off2, 8)], idx_v.at[pl.ds(8, 8)])
        pltpu.async_copy(table_hbm.at[idx_v], rows_v, sem).wait()
        for c in range(d // _LANES):
            sl = pl.ds(c * _LANES, _LANES)
            acc = jnp.zeros((_LANES,), jnp.float32)
            for r in range(_GROUP):
                w_r = jnp.where(base + r < n, 1.0, 0.0)
                acc = acc + rows_v[r, sl] * w_r
            acc_v[sl] = acc
        pltpu.sync_copy(acc_v, out_hbm.at[wid])

    run = pl.kernel(
        body,
        out_type=jax.ShapeDtypeStruct((_NW, d), jnp.float32),
        mesh=mesh,
        scratch_types=[
            pltpu.VMEM((_GROUP,), jnp.int32),
            pltpu.VMEM((_GROUP, d), jnp.float32),
            pltpu.VMEM((d,), jnp.float32),
            pltpu.SemaphoreType.DMA,
        ],
    )
    return run(idx, table)


def _scalar_exp(x):
    return jnp.max(jnp.exp(jnp.full((1, 128), x)))


def _scalar_log(x):
    return jnp.max(jnp.log(jnp.full((1, 128), x)))


def _a_body(nt, vocab, p_ref, wp_ref, bp_ref, wo_hbm, bo_ref, out_ref, *scr):
    bufs = scr[:_NBUF]
    sems = scr[_NBUF:]
    tv = _TV

    def tile_copy(t):
        sz = min(vocab - t * tv, tv)
        return pltpu.make_async_copy(
            wo_hbm.at[pl.ds(t * tv, sz)],
            bufs[t % _NBUF].at[pl.ds(0, sz)],
            sems[t % _NBUF],
        )

    for t in range(min(_NBUF - 1, nt)):
        tile_copy(t).start()

    e = jnp.sum(p_ref[...], axis=0, keepdims=True)
    h = jnp.dot(e, wp_ref[...], preferred_element_type=jnp.float32)
    h = jnp.maximum(h + bp_ref[...], 0.0)
    hT = jnp.transpose(h)
    h8 = jnp.broadcast_to(hT, (hT.shape[0], 8))

    m = jnp.float32(-jnp.inf)
    s = jnp.float32(0.0)
    for t in range(nt):
        if t + _NBUF - 1 < nt:
            tile_copy(t + _NBUF - 1).start()
        tile_copy(t).wait()
        sz = min(vocab - t * tv, tv)
        wo = bufs[t % _NBUF][pl.ds(0, sz), :]
        lt = jnp.dot(wo, h8, preferred_element_type=jnp.float32)
        lr = jnp.transpose(lt)[0:1, :] + bo_ref[:, t * tv:t * tv + sz]
        out_ref[:, t * tv:t * tv + sz] = lr
        m_new = jnp.maximum(m, jnp.max(lr))
        s = s * _scalar_exp(m - m_new) + jnp.sum(jnp.exp(lr - m_new))
        m = m_new

    lse = m + _scalar_log(s)
    out_ref[...] = out_ref[...] - lse


def _tc_mlp_logsoftmax(partials, W_proj, bp2, woT, bo2):
    vocab, d = woT.shape
    nt = -(-vocab // _TV)

    return pl.pallas_call(
        functools.partial(_a_body, nt, vocab),
        in_specs=[
            pl.BlockSpec((_NW, d), lambda: (0, 0)),
            pl.BlockSpec((d, d), lambda: (0, 0)),
            pl.BlockSpec((1, d), lambda: (0, 0)),
            pl.BlockSpec(memory_space=pl.ANY),
            pl.BlockSpec((1, vocab), lambda: (0, 0)),
        ],
        out_specs=pl.BlockSpec((1, vocab), lambda: (0, 0)),
        out_shape=jax.ShapeDtypeStruct((1, vocab), jnp.float32),
        scratch_shapes=(
            [pltpu.VMEM((_TV, d), jnp.float32) for _ in range(_NBUF)]
            + [pltpu.SemaphoreType.DMA for _ in range(_NBUF)]
        ),
    )(partials, W_proj, bp2, woT, bo2)


def kernel(inputs, table, W_proj, b_proj, W_out, b_out):
    idx = inputs.astype(jnp.int32)
    partials = _sc_gather_sum(idx, table)
    return _tc_mlp_logsoftmax(
        partials,
        W_proj,
        b_proj.reshape(1, -1),
        W_out.T,
        b_out.reshape(1, -1),
    )
